# Initial kernel scaffold; baseline (speedup 1.0000x reference)
import functools

import jax
import jax.numpy as jnp
from jax import lax
from jax.experimental import pallas as pl
from jax.experimental.pallas import tpu as pltpu
from jax.experimental.pallas import tpu_sc as plsc

NC_ = 2
NS_ = 16
NW_ = NC_ * NS_
CH_ = 128


def _sc_gather_add(table, idx2d, dst2d, npad, g, ngroup):
  rpt = npad // NS_
  cpt = g * ngroup
  pcc = cpt * NS_
  mesh = plsc.VectorSubcoreMesh(core_axis_name="c", subcore_axis_name="s")

  nfull, tail = divmod(rpt, g * CH_)

  @functools.partial(
      pl.kernel,
      out_type=jax.ShapeDtypeStruct((NC_, npad, 16), jnp.float32),
      mesh=mesh,
      scratch_types=[
          pltpu.VMEM_SHARED((npad, 16), jnp.float32),
          pltpu.VMEM((g, CH_), jnp.int32),
          pltpu.VMEM((g, CH_), jnp.int32),
          pltpu.VMEM((g * CH_, 16), jnp.float32),
          pltpu.SemaphoreType.DMA,
          pltpu.SemaphoreType.DMA,
          pltpu.SemaphoreType.DMA,
      ],
  )
  def k(idx_hbm, dst_hbm, table_hbm, out_hbm, acc, idxb, dstb, rowsb,
        lsem, gsem, ssem):
    c = lax.axis_index("c")
    s = lax.axis_index("s")

    @pl.loop(0, g * CH_)
    def _(i):
      rowsb[i, :] = jnp.zeros((16,), jnp.float32)

    base = s * rpt

    @pl.loop(0, nfull)
    def _(i):
      pltpu.sync_copy(rowsb, acc.at[pl.ds(base + i * (g * CH_), g * CH_)])

    if tail:
      pltpu.sync_copy(rowsb.at[pl.ds(0, tail)],
                      acc.at[pl.ds(base + nfull * (g * CH_), tail)])

    plsc.subcore_barrier()

    chunk0 = c * pcc + s * cpt

    @pl.loop(0, ngroup)
    def _(grp):
      cb = chunk0 + grp * g
      cp_i = pltpu.async_copy(idx_hbm.at[pl.ds(cb, g)], idxb, lsem)
      cp_d = pltpu.async_copy(dst_hbm.at[pl.ds(cb, g)], dstb, lsem)
      cp_i.wait()
      cp_d.wait()
      gathers = [
          pltpu.async_copy(table_hbm.at[idxb.at[j]],
                           rowsb.at[pl.ds(j * CH_, CH_)], gsem)
          for j in range(g)
      ]
      for cp in gathers:
        cp.wait()
      scatters = [
          pltpu.async_copy(rowsb.at[pl.ds(j * CH_, CH_)],
                           acc.at[dstb.at[j]], ssem, add=True)
          for j in range(g)
      ]
      for cp in scatters:
        cp.wait()

    plsc.subcore_barrier()
    pltpu.sync_copy(acc.at[pl.ds(base, rpt)], out_hbm.at[c, pl.ds(base, rpt)])

  return k(idx2d, dst2d, table)


def _sc_degree(dst2d, npad, g, ngroup):
  rpt = npad // NS_
  cpt = g * ngroup
  pcc = cpt * NS_
  mesh = plsc.VectorSubcoreMesh(core_axis_name="c", subcore_axis_name="s")
  nfull, tail = divmod(rpt, CH_)

  @functools.partial(
      pl.kernel,
      out_type=jax.ShapeDtypeStruct((NC_, npad, 16), jnp.float32),
      mesh=mesh,
      scratch_types=[
          pltpu.VMEM_SHARED((npad, 16), jnp.float32),
          pltpu.VMEM((g, CH_), jnp.int32),
          pltpu.VMEM((CH_, 16), jnp.float32),
          pltpu.SemaphoreType.DMA,
          pltpu.SemaphoreType.DMA,
      ],
  )
  def k(dst_hbm, out_hbm, acc, dstb, onesb, lsem, ssem):
    c = lax.axis_index("c")
    s = lax.axis_index("s")

    @pl.loop(0, CH_)
    def _(i):
      onesb[i, :] = jnp.zeros((16,), jnp.float32)

    base = s * rpt

    @pl.loop(0, nfull)
    def _(i):
      pltpu.sync_copy(onesb, acc.at[pl.ds(base + i * CH_, CH_)])

    if tail:
      pltpu.sync_copy(onesb.at[pl.ds(0, tail)],
                      acc.at[pl.ds(base + nfull * CH_, tail)])

    @pl.loop(0, CH_)
    def _(i):
      onesb[i, :] = jnp.full((16,), 1.0, jnp.float32)

    plsc.subcore_barrier()

    chunk0 = c * pcc + s * cpt

    @pl.loop(0, ngroup)
    def _(grp):
      cb = chunk0 + grp * g
      pltpu.async_copy(dst_hbm.at[pl.ds(cb, g)], dstb, lsem).wait()
      scatters = [
          pltpu.async_copy(onesb, acc.at[dstb.at[j]], ssem, add=True)
          for j in range(g)
      ]
      for cp in scatters:
        cp.wait()

    plsc.subcore_barrier()
    pltpu.sync_copy(acc.at[pl.ds(base, rpt)], out_hbm.at[c, pl.ds(base, rpt)])

  return k(dst2d)


def _tc_layer1(degp, s1p, root1, bias1, w2cat, n, blk):
  grid = n // blk

  def body(degp_ref, s1p_ref, root1_ref, bias1_ref, w2_ref,
           h_ref, t2_ref, inv_ref):
    deg = degp_ref[0] + degp_ref[1]
    inv = 1.0 / jnp.maximum(deg, 1.0)
    s1 = (s1p_ref[0] + s1p_ref[1]) * inv
    h = jnp.maximum(s1 + root1_ref[...] + bias1_ref[...], 0.0)
    h_ref[...] = h
    inv_ref[...] = inv
    t2_ref[...] = jax.lax.dot_general(
        h, w2_ref[...], (((1,), (0,)), ((), ())),
        preferred_element_type=jnp.float32,
        precision=jax.lax.Precision.HIGHEST)

  return pl.pallas_call(
      body,
      grid=(grid,),
      in_specs=[
          pl.BlockSpec((NC_, blk, 16), lambda i: (0, i, 0)),
          pl.BlockSpec((NC_, blk, 16), lambda i: (0, i, 0)),
          pl.BlockSpec((blk, 16), lambda i: (i, 0)),
          pl.BlockSpec((1, 16), lambda i: (0, 0)),
          pl.BlockSpec((16, 128), lambda i: (0, 0)),
      ],
      out_specs=[
          pl.BlockSpec((blk, 16), lambda i: (i, 0)),
          pl.BlockSpec((blk, 128), lambda i: (i, 0)),
          pl.BlockSpec((blk, 16), lambda i: (i, 0)),
      ],
      out_shape=[
          jax.ShapeDtypeStruct((n, 16), jnp.float32),
          jax.ShapeDtypeStruct((n, 128), jnp.float32),
          jax.ShapeDtypeStruct((n, 16), jnp.float32),
      ],
  )(degp, s1p, root1, bias1, w2cat)


def _tc_layer2(s2p, inv16, h, root2, bias2, n, blk):
  grid = n // blk

  def body(s2p_ref, inv_ref, h_ref, root2_ref, bias2_ref, out_ref):
    z = (s2p_ref[0] + s2p_ref[1]) * inv_ref[...]
    z = z + jax.lax.dot_general(
        h_ref[...], root2_ref[...], (((1,), (0,)), ((), ())),
        preferred_element_type=jnp.float32,
        precision=jax.lax.Precision.HIGHEST)
    z = z + bias2_ref[...]
    m = jnp.max(z, axis=1, keepdims=True)
    e = jnp.exp(z - m)
    out_ref[...] = z - m - jnp.log(jnp.sum(e, axis=1, keepdims=True))

  return pl.pallas_call(
      body,
      grid=(grid,),
      in_specs=[
          pl.BlockSpec((NC_, blk, 16), lambda i: (0, i, 0)),
          pl.BlockSpec((blk, 16), lambda i: (i, 0)),
          pl.BlockSpec((blk, 16), lambda i: (i, 0)),
          pl.BlockSpec((16, 16), lambda i: (0, 0)),
          pl.BlockSpec((1, 16), lambda i: (0, 0)),
      ],
      out_specs=pl.BlockSpec((blk, 16), lambda i: (i, 0)),
      out_shape=jax.ShapeDtypeStruct((n, 16), jnp.float32),
  )(s2p, inv16, h, root2, bias2)


def kernel(edge_index, edge_type, edge_norm, weight1, root1, bias1,
           weight2, root2, bias2):
  del edge_norm
  r, n, c1 = weight1.shape
  nc2 = weight2.shape[2]
  e = edge_type.shape[0]

  g = 10
  ngroup = 40
  cpt = g * ngroup
  e_pad = NW_ * cpt * CH_
  npad = n + CH_

  src = edge_index[0].astype(jnp.int32)
  dst = edge_index[1].astype(jnp.int32)
  et = edge_type.astype(jnp.int32)

  pad_n = e_pad - e
  pad_iota = lax.iota(jnp.int32, pad_n)
  idx1 = jnp.concatenate([et * n + src, pad_iota % n]).reshape(-1, CH_)
  idx2 = jnp.concatenate([src * r + et, pad_iota % n]).reshape(-1, CH_)
  dstp = jnp.concatenate([dst, n + pad_iota % CH_]).reshape(-1, CH_)

  degp = _sc_degree(dstp, npad, 2 * g, ngroup // 2)

  table1 = weight1.reshape(r * n, c1)
  s1p = _sc_gather_add(table1, idx1, dstp, npad, g, ngroup)

  blk = 2000
  w2cat = jnp.transpose(weight2, (1, 0, 2)).reshape(c1, r * nc2)
  h, t2, inv16 = _tc_layer1(degp[:, :n], s1p[:, :n], root1,
                            bias1.reshape(1, 16), w2cat, n, blk)

  table2 = t2.reshape(n * r, nc2)
  s2p = _sc_gather_add(table2, idx2, dstp, npad, g, ngroup)

  return _tc_layer2(s2p[:, :n], inv16, h, root2, bias2.reshape(1, 16), n, blk)


# speedup vs baseline: 45.5135x; 45.5135x over previous
import functools

import jax
import jax.numpy as jnp
from jax import lax
from jax.experimental import pallas as pl
from jax.experimental.pallas import tpu as pltpu
from jax.experimental.pallas import tpu_sc as plsc

NC_ = 2
NS_ = 16
NW_ = NC_ * NS_
CH_ = 128

_SC_PARAMS = pltpu.CompilerParams(use_tc_tiling_on_sc=False)


def _sc_gather_add(table, idx2d, dst2d, npad, g, ngroup):
  rpt = npad // NS_
  cpt = g * ngroup
  pcc = cpt * NS_
  mesh = plsc.VectorSubcoreMesh(core_axis_name="c", subcore_axis_name="s")

  nfull, tail = divmod(rpt, g * CH_)

  @functools.partial(
      pl.kernel,
      out_type=jax.ShapeDtypeStruct((NC_, npad, 16), jnp.float32),
      mesh=mesh,
      scratch_types=[
          pltpu.VMEM_SHARED((npad, 16), jnp.float32),
          pltpu.VMEM((g, CH_), jnp.int32),
          pltpu.VMEM((g, CH_), jnp.int32),
          pltpu.VMEM((g * CH_, 16), jnp.float32),
          pltpu.SemaphoreType.DMA,
          pltpu.SemaphoreType.DMA,
          pltpu.SemaphoreType.DMA,
      ],
      compiler_params=_SC_PARAMS,
  )
  def k(idx_hbm, dst_hbm, table_hbm, out_hbm, acc, idxb, dstb, rowsb,
        lsem, gsem, ssem):
    c = lax.axis_index("c")
    s = lax.axis_index("s")

    @pl.loop(0, g * CH_)
    def _(i):
      rowsb[i, :] = jnp.zeros((16,), jnp.float32)

    base = s * rpt

    @pl.loop(0, nfull)
    def _(i):
      pltpu.sync_copy(rowsb, acc.at[pl.ds(base + i * (g * CH_), g * CH_)])

    if tail:
      pltpu.sync_copy(rowsb.at[pl.ds(0, tail)],
                      acc.at[pl.ds(base + nfull * (g * CH_), tail)])

    plsc.subcore_barrier()

    chunk0 = c * pcc + s * cpt

    @pl.loop(0, ngroup)
    def _(grp):
      cb = chunk0 + grp * g
      cp_i = pltpu.async_copy(idx_hbm.at[pl.ds(cb, g)], idxb, lsem)
      cp_d = pltpu.async_copy(dst_hbm.at[pl.ds(cb, g)], dstb, lsem)
      cp_i.wait()
      cp_d.wait()
      gathers = [
          pltpu.async_copy(table_hbm.at[idxb.at[j]],
                           rowsb.at[pl.ds(j * CH_, CH_)], gsem)
          for j in range(g)
      ]
      for cp in gathers:
        cp.wait()
      scatters = [
          pltpu.async_copy(rowsb.at[pl.ds(j * CH_, CH_)],
                           acc.at[dstb.at[j]], ssem, add=True)
          for j in range(g)
      ]
      for cp in scatters:
        cp.wait()

    plsc.subcore_barrier()
    pltpu.sync_copy(acc.at[pl.ds(base, rpt)], out_hbm.at[c, pl.ds(base, rpt)])

  return k(idx2d, dst2d, table)


def _sc_degree(dst2d, npad, g, ngroup):
  rpt = npad // NS_
  cpt = g * ngroup
  pcc = cpt * NS_
  mesh = plsc.VectorSubcoreMesh(core_axis_name="c", subcore_axis_name="s")
  nfull, tail = divmod(rpt, CH_)

  @functools.partial(
      pl.kernel,
      out_type=jax.ShapeDtypeStruct((NC_, npad, 16), jnp.float32),
      mesh=mesh,
      scratch_types=[
          pltpu.VMEM_SHARED((npad, 16), jnp.float32),
          pltpu.VMEM((g, CH_), jnp.int32),
          pltpu.VMEM((CH_, 16), jnp.float32),
          pltpu.SemaphoreType.DMA,
          pltpu.SemaphoreType.DMA,
      ],
      compiler_params=_SC_PARAMS,
  )
  def k(dst_hbm, out_hbm, acc, dstb, onesb, lsem, ssem):
    c = lax.axis_index("c")
    s = lax.axis_index("s")

    @pl.loop(0, CH_)
    def _(i):
      onesb[i, :] = jnp.zeros((16,), jnp.float32)

    base = s * rpt

    @pl.loop(0, nfull)
    def _(i):
      pltpu.sync_copy(onesb, acc.at[pl.ds(base + i * CH_, CH_)])

    if tail:
      pltpu.sync_copy(onesb.at[pl.ds(0, tail)],
                      acc.at[pl.ds(base + nfull * CH_, tail)])

    @pl.loop(0, CH_)
    def _(i):
      onesb[i, :] = jnp.full((16,), 1.0, jnp.float32)

    plsc.subcore_barrier()

    chunk0 = c * pcc + s * cpt

    @pl.loop(0, ngroup)
    def _(grp):
      cb = chunk0 + grp * g
      pltpu.async_copy(dst_hbm.at[pl.ds(cb, g)], dstb, lsem).wait()
      scatters = [
          pltpu.async_copy(onesb, acc.at[dstb.at[j]], ssem, add=True)
          for j in range(g)
      ]
      for cp in scatters:
        cp.wait()

    plsc.subcore_barrier()
    pltpu.sync_copy(acc.at[pl.ds(base, rpt)], out_hbm.at[c, pl.ds(base, rpt)])

  return k(dst2d)


def _tc_layer1(degp, s1p, root1, bias1, w2cat, n, blk):
  grid = n // blk

  def body(degp_ref, s1p_ref, root1_ref, bias1_ref, w2_ref,
           h_ref, t2_ref, inv_ref):
    deg = degp_ref[0] + degp_ref[1]
    inv = 1.0 / jnp.maximum(deg, 1.0)
    s1 = (s1p_ref[0] + s1p_ref[1]) * inv
    h = jnp.maximum(s1 + root1_ref[...] + bias1_ref[...], 0.0)
    h_ref[...] = h
    inv_ref[...] = inv
    t2_ref[...] = jax.lax.dot_general(
        h, w2_ref[...], (((1,), (0,)), ((), ())),
        preferred_element_type=jnp.float32,
        precision=jax.lax.Precision.HIGHEST)

  return pl.pallas_call(
      body,
      grid=(grid,),
      in_specs=[
          pl.BlockSpec((NC_, blk, 16), lambda i: (0, i, 0)),
          pl.BlockSpec((NC_, blk, 16), lambda i: (0, i, 0)),
          pl.BlockSpec((blk, 16), lambda i: (i, 0)),
          pl.BlockSpec((1, 16), lambda i: (0, 0)),
          pl.BlockSpec((16, 128), lambda i: (0, 0)),
      ],
      out_specs=[
          pl.BlockSpec((blk, 16), lambda i: (i, 0)),
          pl.BlockSpec((blk, 128), lambda i: (i, 0)),
          pl.BlockSpec((blk, 16), lambda i: (i, 0)),
      ],
      out_shape=[
          jax.ShapeDtypeStruct((n, 16), jnp.float32),
          jax.ShapeDtypeStruct((n, 128), jnp.float32),
          jax.ShapeDtypeStruct((n, 16), jnp.float32),
      ],
  )(degp, s1p, root1, bias1, w2cat)


def _tc_layer2(s2p, inv16, h, root2, bias2, n, blk):
  grid = n // blk

  def body(s2p_ref, inv_ref, h_ref, root2_ref, bias2_ref, out_ref):
    z = (s2p_ref[0] + s2p_ref[1]) * inv_ref[...]
    z = z + jax.lax.dot_general(
        h_ref[...], root2_ref[...], (((1,), (0,)), ((), ())),
        preferred_element_type=jnp.float32,
        precision=jax.lax.Precision.HIGHEST)
    z = z + bias2_ref[...]
    m = jnp.max(z, axis=1, keepdims=True)
    e = jnp.exp(z - m)
    out_ref[...] = z - m - jnp.log(jnp.sum(e, axis=1, keepdims=True))

  return pl.pallas_call(
      body,
      grid=(grid,),
      in_specs=[
          pl.BlockSpec((NC_, blk, 16), lambda i: (0, i, 0)),
          pl.BlockSpec((blk, 16), lambda i: (i, 0)),
          pl.BlockSpec((blk, 16), lambda i: (i, 0)),
          pl.BlockSpec((16, 16), lambda i: (0, 0)),
          pl.BlockSpec((1, 16), lambda i: (0, 0)),
      ],
      out_specs=pl.BlockSpec((blk, 16), lambda i: (i, 0)),
      out_shape=jax.ShapeDtypeStruct((n, 16), jnp.float32),
  )(s2p, inv16, h, root2, bias2)


def kernel(edge_index, edge_type, edge_norm, weight1, root1, bias1,
           weight2, root2, bias2):
  del edge_norm
  r, n, c1 = weight1.shape
  nc2 = weight2.shape[2]
  e = edge_type.shape[0]

  g = 8
  ngroup = 50
  cpt = g * ngroup
  e_pad = NW_ * cpt * CH_
  npad = -(-(n + 1) // CH_) * CH_
  ndum = npad - n

  src = edge_index[0].astype(jnp.int32)
  dst = edge_index[1].astype(jnp.int32)
  et = edge_type.astype(jnp.int32)

  pad_n = e_pad - e
  pad_iota = lax.iota(jnp.int32, pad_n)
  idx1 = jnp.concatenate([et * n + src, pad_iota % n]).reshape(-1, CH_)
  idx2 = jnp.concatenate([src * r + et, pad_iota % n]).reshape(-1, CH_)
  dstp = jnp.concatenate([dst, n + pad_iota % ndum]).reshape(-1, CH_)

  degp = _sc_degree(dstp, npad, 2 * g, ngroup // 2)

  table1 = weight1.reshape(r * n, c1)
  s1p = _sc_gather_add(table1, idx1, dstp, npad, g, ngroup)

  blk = 2000
  w2cat = jnp.transpose(weight2, (1, 0, 2)).reshape(c1, r * nc2)
  h, t2, inv16 = _tc_layer1(degp[:, :n], s1p[:, :n], root1,
                            bias1.reshape(1, 16), w2cat, n, blk)

  table2 = t2.reshape(n * r, nc2)
  s2p = _sc_gather_add(table2, idx2, dstp, npad, g, ngroup)

  return _tc_layer2(s2p[:, :n], inv16, h, root2, bias2.reshape(1, 16), n, blk)


# speedup vs baseline: 54.4244x; 1.1958x over previous
import functools

import jax
import jax.numpy as jnp
from jax import lax
from jax.experimental import pallas as pl
from jax.experimental.pallas import tpu as pltpu
from jax.experimental.pallas import tpu_sc as plsc

NC_ = 2
NS_ = 16
NW_ = NC_ * NS_
CH_ = 128

G_ = 4
GPS_ = 14
NSB_ = 7
CPT_ = G_ * GPS_ * NSB_

_SC_PARAMS = pltpu.CompilerParams(use_tc_tiling_on_sc=False)


def _zero_rows(buf, nrows):
  @pl.loop(0, nrows)
  def _(i):
    buf[i, :] = jnp.zeros((16,), jnp.float32)


def _zero_acc_slice(acc, zsrc, base, rpt):
  zrows = zsrc.shape[0]
  nfull, tail = divmod(rpt, zrows)

  @pl.loop(0, nfull)
  def _(i):
    pltpu.sync_copy(zsrc, acc.at[pl.ds(base + i * zrows, zrows)])

  if tail:
    pltpu.sync_copy(zsrc.at[pl.ds(0, tail)],
                    acc.at[pl.ds(base + nfull * zrows, tail)])


def _sc_gather_add(table, idx2d, dst2d, npad):
  rpt = npad // NS_
  spc = G_ * GPS_
  pcc = CPT_ * NS_
  grows = G_ * CH_
  mesh = plsc.VectorSubcoreMesh(core_axis_name="c", subcore_axis_name="s")

  @functools.partial(
      pl.kernel,
      out_type=jax.ShapeDtypeStruct((NC_, npad, 16), jnp.float32),
      mesh=mesh,
      scratch_types=[
          pltpu.VMEM_SHARED((npad, 16), jnp.float32),
          pltpu.VMEM((spc, CH_), jnp.int32),
          pltpu.VMEM((spc, CH_), jnp.int32),
          pltpu.VMEM((grows, 16), jnp.float32),
          pltpu.VMEM((grows, 16), jnp.float32),
          pltpu.SemaphoreType.DMA,
          pltpu.SemaphoreType.DMA,
          pltpu.SemaphoreType.DMA,
      ],
      compiler_params=_SC_PARAMS,
  )
  def k(idx_hbm, dst_hbm, table_hbm, out_hbm, acc, idxa, dsta, rows0, rows1,
        lsem, gsem, ssem):
    c = lax.axis_index("c")
    s = lax.axis_index("s")
    rows = (rows0, rows1)

    _zero_rows(rows0, grows)
    base = s * rpt
    _zero_acc_slice(acc, rows0, base, rpt)
    plsc.subcore_barrier()

    chunk0 = c * pcc + s * CPT_

    def issue_gathers(t, b):
      for j in range(G_):
        pltpu.async_copy(table_hbm.at[idxa.at[t * G_ + j]],
                         rows[b].at[pl.ds(j * CH_, CH_)], gsem)

    def issue_scatters(t, b):
      for j in range(G_):
        pltpu.async_copy(rows[b].at[pl.ds(j * CH_, CH_)],
                         acc.at[dsta.at[t * G_ + j]], ssem, add=True)

    def drain_g(b):
      pltpu.make_async_copy(table_hbm.at[pl.ds(0, grows)], rows[b], gsem).wait()

    def drain_s(b):
      pltpu.make_async_copy(rows[b], acc.at[pl.ds(0, grows)], ssem).wait()

    @pl.loop(0, NSB_)
    def _(sb):
      cb = chunk0 + sb * spc
      cpi = pltpu.async_copy(idx_hbm.at[pl.ds(cb, spc)], idxa, lsem)
      cpd = pltpu.async_copy(dst_hbm.at[pl.ds(cb, spc)], dsta, lsem)
      cpi.wait()
      cpd.wait()

      @pl.loop(0, GPS_ // 2)
      def _(p):
        t0 = 2 * p

        @pl.when(p > 0)
        def _():
          drain_g(1)
          issue_scatters(t0 - 1, 1)
          drain_s(0)
        issue_gathers(t0, 0)

        drain_g(0)
        issue_scatters(t0, 0)

        @pl.when(p > 0)
        def _():
          drain_s(1)
        issue_gathers(t0 + 1, 1)

      drain_g(1)
      issue_scatters(GPS_ - 1, 1)
      drain_s(0)
      drain_s(1)

    plsc.subcore_barrier()
    pltpu.sync_copy(acc.at[pl.ds(base, rpt)], out_hbm.at[c, pl.ds(base, rpt)])

  return k(idx2d, dst2d, table)


def _sc_degree(dst2d, npad):
  rpt = npad // NS_
  pcc = CPT_ * NS_
  grows = G_ * CH_
  mesh = plsc.VectorSubcoreMesh(core_axis_name="c", subcore_axis_name="s")

  @functools.partial(
      pl.kernel,
      out_type=jax.ShapeDtypeStruct((NC_, npad, 16), jnp.float32),
      mesh=mesh,
      scratch_types=[
          pltpu.VMEM_SHARED((npad, 16), jnp.float32),
          pltpu.VMEM((G_ * GPS_, CH_), jnp.int32),
          pltpu.VMEM((grows, 16), jnp.float32),
          pltpu.SemaphoreType.DMA,
          pltpu.SemaphoreType.DMA,
      ],
      compiler_params=_SC_PARAMS,
  )
  def k(dst_hbm, out_hbm, acc, dsta, onesb, lsem, ssem):
    c = lax.axis_index("c")
    s = lax.axis_index("s")
    spc = G_ * GPS_

    _zero_rows(onesb, grows)
    base = s * rpt
    _zero_acc_slice(acc, onesb, base, rpt)

    @pl.loop(0, grows)
    def _(i):
      onesb[i, :] = jnp.full((16,), 1.0, jnp.float32)

    plsc.subcore_barrier()
    chunk0 = c * pcc + s * CPT_

    def drain_s():
      pltpu.make_async_copy(onesb, acc.at[pl.ds(0, grows)], ssem).wait()

    @pl.loop(0, NSB_)
    def _(sb):
      pltpu.async_copy(dst_hbm.at[pl.ds(chunk0 + sb * spc, spc)], dsta,
                       lsem).wait()

      @pl.loop(0, GPS_)
      def _(t):
        @pl.when(t >= 2)
        def _():
          drain_s()
        for j in range(G_):
          pltpu.async_copy(onesb.at[pl.ds(j * CH_, CH_)],
                           acc.at[dsta.at[t * G_ + j]], ssem, add=True)

      drain_s()
      drain_s()

    plsc.subcore_barrier()
    pltpu.sync_copy(acc.at[pl.ds(base, rpt)], out_hbm.at[c, pl.ds(base, rpt)])

  return k(dst2d)


def _tc_layer1(degp, s1p, root1, bias1, w2cat, n, blk):
  grid = n // blk

  def body(degp_ref, s1p_ref, root1_ref, bias1_ref, w2_ref,
           h_ref, t2_ref, inv_ref):
    deg = degp_ref[0] + degp_ref[1]
    inv = 1.0 / jnp.maximum(deg, 1.0)
    s1 = (s1p_ref[0] + s1p_ref[1]) * inv
    h = jnp.maximum(s1 + root1_ref[...] + bias1_ref[...], 0.0)
    h_ref[...] = h
    inv_ref[...] = inv
    t2_ref[...] = jax.lax.dot_general(
        h, w2_ref[...], (((1,), (0,)), ((), ())),
        preferred_element_type=jnp.float32,
        precision=jax.lax.Precision.HIGHEST)

  return pl.pallas_call(
      body,
      grid=(grid,),
      in_specs=[
          pl.BlockSpec((NC_, blk, 16), lambda i: (0, i, 0)),
          pl.BlockSpec((NC_, blk, 16), lambda i: (0, i, 0)),
          pl.BlockSpec((blk, 16), lambda i: (i, 0)),
          pl.BlockSpec((1, 16), lambda i: (0, 0)),
          pl.BlockSpec((16, 128), lambda i: (0, 0)),
      ],
      out_specs=[
          pl.BlockSpec((blk, 16), lambda i: (i, 0)),
          pl.BlockSpec((blk, 128), lambda i: (i, 0)),
          pl.BlockSpec((blk, 16), lambda i: (i, 0)),
      ],
      out_shape=[
          jax.ShapeDtypeStruct((n, 16), jnp.float32),
          jax.ShapeDtypeStruct((n, 128), jnp.float32),
          jax.ShapeDtypeStruct((n, 16), jnp.float32),
      ],
  )(degp, s1p, root1, bias1, w2cat)


def _tc_layer2(s2p, inv16, h, root2, bias2, n, blk):
  grid = n // blk

  def body(s2p_ref, inv_ref, h_ref, root2_ref, bias2_ref, out_ref):
    z = (s2p_ref[0] + s2p_ref[1]) * inv_ref[...]
    z = z + jax.lax.dot_general(
        h_ref[...], root2_ref[...], (((1,), (0,)), ((), ())),
        preferred_element_type=jnp.float32,
        precision=jax.lax.Precision.HIGHEST)
    z = z + bias2_ref[...]
    m = jnp.max(z, axis=1, keepdims=True)
    e = jnp.exp(z - m)
    out_ref[...] = z - m - jnp.log(jnp.sum(e, axis=1, keepdims=True))

  return pl.pallas_call(
      body,
      grid=(grid,),
      in_specs=[
          pl.BlockSpec((NC_, blk, 16), lambda i: (0, i, 0)),
          pl.BlockSpec((blk, 16), lambda i: (i, 0)),
          pl.BlockSpec((blk, 16), lambda i: (i, 0)),
          pl.BlockSpec((16, 16), lambda i: (0, 0)),
          pl.BlockSpec((1, 16), lambda i: (0, 0)),
      ],
      out_specs=pl.BlockSpec((blk, 16), lambda i: (i, 0)),
      out_shape=jax.ShapeDtypeStruct((n, 16), jnp.float32),
  )(s2p, inv16, h, root2, bias2)


def kernel(edge_index, edge_type, edge_norm, weight1, root1, bias1,
           weight2, root2, bias2):
  del edge_norm
  r, n, c1 = weight1.shape
  nc2 = weight2.shape[2]
  e = edge_type.shape[0]

  e_pad = NW_ * CPT_ * CH_
  npad = -(-(n + 1) // CH_) * CH_
  ndum = npad - n

  src = edge_index[0].astype(jnp.int32)
  dst = edge_index[1].astype(jnp.int32)
  et = edge_type.astype(jnp.int32)

  pad_n = e_pad - e
  pad_iota = lax.iota(jnp.int32, pad_n)
  idx1 = jnp.concatenate([et * n + src, pad_iota % n]).reshape(-1, CH_)
  idx2 = jnp.concatenate([src * r + et, pad_iota % n]).reshape(-1, CH_)
  dstp = jnp.concatenate([dst, n + pad_iota % ndum]).reshape(-1, CH_)

  degp = _sc_degree(dstp, npad)

  table1 = weight1.reshape(r * n, c1)
  s1p = _sc_gather_add(table1, idx1, dstp, npad)

  blk = 2000
  w2cat = jnp.transpose(weight2, (1, 0, 2)).reshape(c1, r * nc2)
  h, t2, inv16 = _tc_layer1(degp, s1p, root1, bias1.reshape(1, 16), w2cat,
                            n, blk)

  table2 = t2.reshape(n * r, nc2)
  s2p = _sc_gather_add(table2, idx2, dstp, npad)

  return _tc_layer2(s2p, inv16, h, root2, bias2.reshape(1, 16), n, blk)


# speedup vs baseline: 55.2842x; 1.0158x over previous
import functools

import jax
import jax.numpy as jnp
from jax import lax
from jax.experimental import pallas as pl
from jax.experimental.pallas import tpu as pltpu
from jax.experimental.pallas import tpu_sc as plsc

NC_ = 2
NS_ = 16
NW_ = NC_ * NS_
CH_ = 128

G_ = 4
GPS_ = 14
NSB_ = 7
CPT_ = G_ * GPS_ * NSB_

_SC_PARAMS = pltpu.CompilerParams(use_tc_tiling_on_sc=False)


def _zero_rows(buf, nrows):
  @pl.loop(0, nrows)
  def _(i):
    buf[i, :] = jnp.zeros((16,), jnp.float32)


def _zero_acc_slice(acc, zsrc, base, rpt):
  zrows = zsrc.shape[0]
  nfull, tail = divmod(rpt, zrows)

  @pl.loop(0, nfull)
  def _(i):
    pltpu.sync_copy(zsrc, acc.at[pl.ds(base + i * zrows, zrows)])

  if tail:
    pltpu.sync_copy(zsrc.at[pl.ds(0, tail)],
                    acc.at[pl.ds(base + nfull * zrows, tail)])


def _sc_gather_add(table, idx2d, dst2d, npad):
  rpt = npad // NS_
  spc = G_ * GPS_
  pcc = CPT_ * NS_
  grows = G_ * CH_
  mesh = plsc.VectorSubcoreMesh(core_axis_name="c", subcore_axis_name="s")

  @functools.partial(
      pl.kernel,
      out_type=jax.ShapeDtypeStruct((NC_, npad, 16), jnp.float32),
      mesh=mesh,
      scratch_types=[
          pltpu.VMEM_SHARED((npad, 16), jnp.float32),
          pltpu.VMEM((spc, CH_), jnp.int32),
          pltpu.VMEM((spc, CH_), jnp.int32),
          pltpu.VMEM((grows, 16), jnp.float32),
          pltpu.VMEM((grows, 16), jnp.float32),
          pltpu.SemaphoreType.DMA,
          pltpu.SemaphoreType.DMA,
          pltpu.SemaphoreType.DMA,
      ],
      compiler_params=_SC_PARAMS,
  )
  def k(idx_hbm, dst_hbm, table_hbm, out_hbm, acc, idxa, dsta, rows0, rows1,
        lsem, gsem, ssem):
    c = lax.axis_index("c")
    s = lax.axis_index("s")
    rows = (rows0, rows1)

    _zero_rows(rows0, grows)
    base = s * rpt
    _zero_acc_slice(acc, rows0, base, rpt)
    plsc.subcore_barrier()

    chunk0 = c * pcc + s * CPT_

    def issue_gathers(t, b):
      for j in range(G_):
        pltpu.async_copy(table_hbm.at[idxa.at[t * G_ + j]],
                         rows[b].at[pl.ds(j * CH_, CH_)], gsem)

    def issue_scatters(t, b):
      for j in range(G_):
        pltpu.async_copy(rows[b].at[pl.ds(j * CH_, CH_)],
                         acc.at[dsta.at[t * G_ + j]], ssem, add=True)

    def drain_g(b):
      pltpu.make_async_copy(table_hbm.at[pl.ds(0, grows)], rows[b], gsem).wait()

    def drain_s(b):
      pltpu.make_async_copy(rows[b], acc.at[pl.ds(0, grows)], ssem).wait()

    @pl.loop(0, NSB_)
    def _(sb):
      cb = chunk0 + sb * spc
      cpi = pltpu.async_copy(idx_hbm.at[pl.ds(cb, spc)], idxa, lsem)
      cpd = pltpu.async_copy(dst_hbm.at[pl.ds(cb, spc)], dsta, lsem)
      cpi.wait()
      cpd.wait()

      @pl.loop(0, GPS_, step=2)
      def _(base_t):
        for b2 in (0, 1):
          t = base_t + b2
          if b2 == 0:
            @pl.when(base_t > 0)
            def _():
              drain_g(1)
              issue_scatters(t - 1, 1)
              drain_s(0)
          else:
            drain_g(0)
            issue_scatters(t - 1, 0)

            @pl.when(base_t > 0)
            def _():
              drain_s(1)
          issue_gathers(t, b2)

      drain_g(1)
      issue_scatters(GPS_ - 1, 1)
      drain_s(0)
      drain_s(1)

    plsc.subcore_barrier()
    pltpu.sync_copy(acc.at[pl.ds(base, rpt)], out_hbm.at[c, pl.ds(base, rpt)])

  return k(idx2d, dst2d, table)


def _sc_degree(dst2d, npad):
  rpt = npad // NS_
  pcc = CPT_ * NS_
  grows = G_ * CH_
  mesh = plsc.VectorSubcoreMesh(core_axis_name="c", subcore_axis_name="s")

  @functools.partial(
      pl.kernel,
      out_type=jax.ShapeDtypeStruct((NC_, npad, 16), jnp.float32),
      mesh=mesh,
      scratch_types=[
          pltpu.VMEM_SHARED((npad, 16), jnp.float32),
          pltpu.VMEM((G_ * GPS_, CH_), jnp.int32),
          pltpu.VMEM((grows, 16), jnp.float32),
          pltpu.SemaphoreType.DMA,
          pltpu.SemaphoreType.DMA,
      ],
      compiler_params=_SC_PARAMS,
  )
  def k(dst_hbm, out_hbm, acc, dsta, onesb, lsem, ssem):
    c = lax.axis_index("c")
    s = lax.axis_index("s")
    spc = G_ * GPS_

    _zero_rows(onesb, grows)
    base = s * rpt
    _zero_acc_slice(acc, onesb, base, rpt)

    @pl.loop(0, grows)
    def _(i):
      onesb[i, :] = jnp.full((16,), 1.0, jnp.float32)

    plsc.subcore_barrier()
    chunk0 = c * pcc + s * CPT_

    def drain_s():
      pltpu.make_async_copy(onesb, acc.at[pl.ds(0, grows)], ssem).wait()

    @pl.loop(0, NSB_)
    def _(sb):
      pltpu.async_copy(dst_hbm.at[pl.ds(chunk0 + sb * spc, spc)], dsta,
                       lsem).wait()

      @pl.loop(0, GPS_)
      def _(t):
        @pl.when(t >= 2)
        def _():
          drain_s()
        for j in range(G_):
          pltpu.async_copy(onesb.at[pl.ds(j * CH_, CH_)],
                           acc.at[dsta.at[t * G_ + j]], ssem, add=True)

      drain_s()
      drain_s()

    plsc.subcore_barrier()
    pltpu.sync_copy(acc.at[pl.ds(base, rpt)], out_hbm.at[c, pl.ds(base, rpt)])

  return k(dst2d)


def _tc_table1(w1t, r, n, c1, blk):
  grid = -(-n // blk)

  def body(w_ref, t_ref):
    for et in range(r):
      t_ref[:, et * c1:(et + 1) * c1] = jnp.transpose(w_ref[et], (1, 0))

  return pl.pallas_call(
      body,
      grid=(grid,),
      in_specs=[pl.BlockSpec((r, c1, blk), lambda i: (0, 0, i))],
      out_specs=pl.BlockSpec((blk, r * c1), lambda i: (i, 0)),
      out_shape=jax.ShapeDtypeStruct((n, r * c1), jnp.float32),
  )(w1t)


def _tc_layer1(degp, s1p, r1t, bias1, w2cat, n, npad, blk):
  grid = -(-n // blk)

  def body(degp_ref, s1p_ref, r1t_ref, bias1_ref, w2_ref,
           h_ref, t2_ref, inv_ref):
    deg = degp_ref[0] + degp_ref[1]
    inv = 1.0 / jnp.maximum(deg, 1.0)
    s1 = (s1p_ref[0] + s1p_ref[1]) * inv
    root1 = jnp.transpose(r1t_ref[...], (1, 0))
    h = jnp.maximum(s1 + root1 + bias1_ref[...], 0.0)
    h_ref[...] = h
    inv_ref[...] = inv
    t2_ref[...] = jax.lax.dot_general(
        h, w2_ref[...], (((1,), (0,)), ((), ())),
        preferred_element_type=jnp.float32,
        precision=jax.lax.Precision.HIGHEST)

  return pl.pallas_call(
      body,
      grid=(grid,),
      in_specs=[
          pl.BlockSpec((NC_, blk, 16), lambda i: (0, i, 0)),
          pl.BlockSpec((NC_, blk, 16), lambda i: (0, i, 0)),
          pl.BlockSpec((16, blk), lambda i: (0, i)),
          pl.BlockSpec((1, 16), lambda i: (0, 0)),
          pl.BlockSpec((16, 128), lambda i: (0, 0)),
      ],
      out_specs=[
          pl.BlockSpec((blk, 16), lambda i: (i, 0)),
          pl.BlockSpec((blk, 128), lambda i: (i, 0)),
          pl.BlockSpec((blk, 16), lambda i: (i, 0)),
      ],
      out_shape=[
          jax.ShapeDtypeStruct((n, 16), jnp.float32),
          jax.ShapeDtypeStruct((n, 128), jnp.float32),
          jax.ShapeDtypeStruct((n, 16), jnp.float32),
      ],
  )(degp, s1p, r1t, bias1, w2cat)


def _tc_layer2(s2p, inv16, h, root2, bias2, n, blk):
  grid = -(-n // blk)

  def body(s2p_ref, inv_ref, h_ref, root2_ref, bias2_ref, out_ref):
    z = (s2p_ref[0] + s2p_ref[1]) * inv_ref[...]
    z = z + jax.lax.dot_general(
        h_ref[...], root2_ref[...], (((1,), (0,)), ((), ())),
        preferred_element_type=jnp.float32,
        precision=jax.lax.Precision.HIGHEST)
    z = z + bias2_ref[...]
    m = jnp.max(z, axis=1, keepdims=True)
    e = jnp.exp(z - m)
    out_ref[...] = z - m - jnp.log(jnp.sum(e, axis=1, keepdims=True))

  return pl.pallas_call(
      body,
      grid=(grid,),
      in_specs=[
          pl.BlockSpec((NC_, blk, 16), lambda i: (0, i, 0)),
          pl.BlockSpec((blk, 16), lambda i: (i, 0)),
          pl.BlockSpec((blk, 16), lambda i: (i, 0)),
          pl.BlockSpec((16, 16), lambda i: (0, 0)),
          pl.BlockSpec((1, 16), lambda i: (0, 0)),
      ],
      out_specs=pl.BlockSpec((blk, 16), lambda i: (i, 0)),
      out_shape=jax.ShapeDtypeStruct((n, 16), jnp.float32),
  )(s2p, inv16, h, root2, bias2)


def kernel(edge_index, edge_type, edge_norm, weight1, root1, bias1,
           weight2, root2, bias2):
  del edge_norm
  r, n, c1 = weight1.shape
  nc2 = weight2.shape[2]
  e = edge_type.shape[0]

  e_pad = NW_ * CPT_ * CH_
  npad = -(-(n + 1) // CH_) * CH_
  ndum = npad - n

  src = edge_index[0].astype(jnp.int32)
  dst = edge_index[1].astype(jnp.int32)
  et = edge_type.astype(jnp.int32)

  pad_n = e_pad - e
  pad_iota = lax.iota(jnp.int32, pad_n)
  idxs = jnp.concatenate([src * r + et, pad_iota % n]).reshape(-1, CH_)
  dstp = jnp.concatenate([dst, n + pad_iota % ndum]).reshape(-1, CH_)

  degp = _sc_degree(dstp, npad)

  blk = 2048
  w1t = jnp.transpose(weight1, (0, 2, 1))
  table1 = _tc_table1(w1t, r, n, c1, blk).reshape(n * r, c1)
  s1p = _sc_gather_add(table1, idxs, dstp, npad)

  w2cat = jnp.transpose(weight2, (1, 0, 2)).reshape(c1, r * nc2)
  r1t = jnp.transpose(root1, (1, 0))
  h, t2, inv16 = _tc_layer1(degp, s1p, r1t, bias1.reshape(1, 16), w2cat,
                            n, npad, blk)

  table2 = t2.reshape(n * r, nc2)
  s2p = _sc_gather_add(table2, idxs, dstp, npad)

  return _tc_layer2(s2p, inv16, h, root2, bias2.reshape(1, 16), n, blk)


# speedup vs baseline: 66.3274x; 1.1998x over previous
import functools

import jax
import jax.numpy as jnp
from jax import lax
from jax.experimental import pallas as pl
from jax.experimental.pallas import tpu as pltpu
from jax.experimental.pallas import tpu_sc as plsc

NC_ = 2
NS_ = 16
NW_ = NC_ * NS_
CH_ = 128

G_ = 4
GPS_ = 14
NSB_ = 7
CPT_ = G_ * GPS_ * NSB_

_SC_PARAMS = pltpu.CompilerParams(use_tc_tiling_on_sc=False)


def _zero_rows(buf, nrows):
  @pl.loop(0, nrows)
  def _(i):
    buf[i, :] = jnp.zeros((16,), jnp.float32)


def _zero_acc_slice(acc, zsrc, base, rpt):
  zrows = zsrc.shape[0]
  nfull, tail = divmod(rpt, zrows)

  @pl.loop(0, nfull)
  def _(i):
    pltpu.sync_copy(zsrc, acc.at[pl.ds(base + i * zrows, zrows)])

  if tail:
    pltpu.sync_copy(zsrc.at[pl.ds(0, tail)],
                    acc.at[pl.ds(base + nfull * zrows, tail)])


def _sc_gather_add(table, idx2d, dst2d, npad):
  rpt = npad // NS_
  spc = G_ * GPS_
  pcc = CPT_ * NS_
  grows = G_ * CH_
  mesh = plsc.VectorSubcoreMesh(core_axis_name="c", subcore_axis_name="s")

  @functools.partial(
      pl.kernel,
      out_type=jax.ShapeDtypeStruct((NC_, npad, 16), jnp.float32),
      mesh=mesh,
      scratch_types=[
          pltpu.VMEM_SHARED((npad, 16), jnp.float32),
          pltpu.VMEM((spc, CH_), jnp.int32),
          pltpu.VMEM((spc, CH_), jnp.int32),
          pltpu.VMEM((grows, 16), jnp.float32),
          pltpu.VMEM((grows, 16), jnp.float32),
          pltpu.SemaphoreType.DMA,
          pltpu.SemaphoreType.DMA,
          pltpu.SemaphoreType.DMA,
      ],
      compiler_params=_SC_PARAMS,
  )
  def k(idx_hbm, dst_hbm, table_hbm, out_hbm, acc, idxa, dsta, rows0, rows1,
        lsem, gsem, ssem):
    c = lax.axis_index("c")
    s = lax.axis_index("s")
    rows = (rows0, rows1)

    _zero_rows(rows0, grows)
    base = s * rpt
    _zero_acc_slice(acc, rows0, base, rpt)
    plsc.subcore_barrier()

    chunk0 = c * pcc + s * CPT_

    def issue_gathers(t, b):
      for j in range(G_):
        pltpu.async_copy(table_hbm.at[idxa.at[t * G_ + j]],
                         rows[b].at[pl.ds(j * CH_, CH_)], gsem)

    def issue_scatters(t, b):
      for j in range(G_):
        pltpu.async_copy(rows[b].at[pl.ds(j * CH_, CH_)],
                         acc.at[dsta.at[t * G_ + j]], ssem, add=True)

    def drain_g(b):
      pltpu.make_async_copy(table_hbm.at[pl.ds(0, grows)], rows[b], gsem).wait()

    def drain_s(b):
      pltpu.make_async_copy(rows[b], acc.at[pl.ds(0, grows)], ssem).wait()

    @pl.loop(0, NSB_)
    def _(sb):
      cb = chunk0 + sb * spc
      cpi = pltpu.async_copy(idx_hbm.at[pl.ds(cb, spc)], idxa, lsem)
      cpd = pltpu.async_copy(dst_hbm.at[pl.ds(cb, spc)], dsta, lsem)
      cpi.wait()
      cpd.wait()

      @pl.loop(0, GPS_, step=2)
      def _(base_t):
        for b2 in (0, 1):
          t = base_t + b2
          if b2 == 0:
            @pl.when(base_t > 0)
            def _():
              drain_g(1)
              issue_scatters(t - 1, 1)
              drain_s(0)
          else:
            drain_g(0)
            issue_scatters(t - 1, 0)

            @pl.when(base_t > 0)
            def _():
              drain_s(1)
          issue_gathers(t, b2)

      drain_g(1)
      issue_scatters(GPS_ - 1, 1)
      drain_s(0)
      drain_s(1)

    plsc.subcore_barrier()
    pltpu.sync_copy(acc.at[pl.ds(base, rpt)], out_hbm.at[c, pl.ds(base, rpt)])

  return k(idx2d, dst2d, table)


def _sc_degree(dst2d, npad):
  rpt = npad // NS_
  pcc = CPT_ * NS_
  grows = G_ * CH_
  mesh = plsc.VectorSubcoreMesh(core_axis_name="c", subcore_axis_name="s")

  @functools.partial(
      pl.kernel,
      out_type=jax.ShapeDtypeStruct((NC_, npad, 16), jnp.float32),
      mesh=mesh,
      scratch_types=[
          pltpu.VMEM_SHARED((npad, 16), jnp.float32),
          pltpu.VMEM((G_ * GPS_, CH_), jnp.int32),
          pltpu.VMEM((grows, 16), jnp.float32),
          pltpu.SemaphoreType.DMA,
          pltpu.SemaphoreType.DMA,
      ],
      compiler_params=_SC_PARAMS,
  )
  def k(dst_hbm, out_hbm, acc, dsta, onesb, lsem, ssem):
    c = lax.axis_index("c")
    s = lax.axis_index("s")
    spc = G_ * GPS_

    _zero_rows(onesb, grows)
    base = s * rpt
    _zero_acc_slice(acc, onesb, base, rpt)

    @pl.loop(0, grows)
    def _(i):
      onesb[i, :] = jnp.full((16,), 1.0, jnp.float32)

    plsc.subcore_barrier()
    chunk0 = c * pcc + s * CPT_

    def drain_s():
      pltpu.make_async_copy(onesb, acc.at[pl.ds(0, grows)], ssem).wait()

    @pl.loop(0, NSB_)
    def _(sb):
      pltpu.async_copy(dst_hbm.at[pl.ds(chunk0 + sb * spc, spc)], dsta,
                       lsem).wait()

      @pl.loop(0, GPS_)
      def _(t):
        @pl.when(t >= 2)
        def _():
          drain_s()
        for j in range(G_):
          pltpu.async_copy(onesb.at[pl.ds(j * CH_, CH_)],
                           acc.at[dsta.at[t * G_ + j]], ssem, add=True)

      drain_s()
      drain_s()

    plsc.subcore_barrier()
    pltpu.sync_copy(acc.at[pl.ds(base, rpt)], out_hbm.at[c, pl.ds(base, rpt)])

  return k(dst2d)


def _tc_table1(w128, rc, n, blk):
  grid = -(-n // blk)

  def body(w_ref, t_ref):
    t_ref[...] = jnp.transpose(w_ref[...], (1, 0))

  return pl.pallas_call(
      body,
      grid=(grid,),
      in_specs=[pl.BlockSpec((rc, blk), lambda i: (0, i))],
      out_specs=pl.BlockSpec((blk, rc), lambda i: (i, 0)),
      out_shape=jax.ShapeDtypeStruct((n, rc), jnp.float32),
  )(w128)


def _tc_indices(ei2, et2, r, n, npad, nchunk_pad, blk):
  grid = nchunk_pad // blk
  nchunk = ei2.shape[1]

  def body(ei_ref, et_ref, idx_ref, dst_ref):
    i = pl.program_id(0)
    row = jax.lax.broadcasted_iota(jnp.int32, (blk, 128), 0) + i * blk
    lane = jax.lax.broadcasted_iota(jnp.int32, (blk, 128), 1)
    gid = row * 128 + lane
    real = row < nchunk
    idx_ref[...] = jnp.where(real, ei_ref[0] * r + et_ref[...], gid % n)
    dst_ref[...] = jnp.where(real, ei_ref[1], n + gid % (npad - n))

  return pl.pallas_call(
      body,
      grid=(grid,),
      in_specs=[
          pl.BlockSpec((2, blk, 128), lambda i: (0, i, 0)),
          pl.BlockSpec((blk, 128), lambda i: (i, 0)),
      ],
      out_specs=[
          pl.BlockSpec((blk, 128), lambda i: (i, 0)),
          pl.BlockSpec((blk, 128), lambda i: (i, 0)),
      ],
      out_shape=[
          jax.ShapeDtypeStruct((nchunk_pad, 128), jnp.int32),
          jax.ShapeDtypeStruct((nchunk_pad, 128), jnp.int32),
      ],
  )(ei2, et2)


def _tc_layer1(degp, s1p, r1t, bias1, w2cat, n, npad, blk):
  grid = -(-n // blk)

  def body(degp_ref, s1p_ref, r1t_ref, bias1_ref, w2_ref,
           h_ref, t2_ref, inv_ref):
    deg = degp_ref[0] + degp_ref[1]
    inv = 1.0 / jnp.maximum(deg, 1.0)
    s1 = (s1p_ref[0] + s1p_ref[1]) * inv
    root1 = jnp.transpose(r1t_ref[...], (1, 0))
    h = jnp.maximum(s1 + root1 + bias1_ref[...], 0.0)
    h_ref[...] = h
    inv_ref[...] = inv
    t2_ref[...] = jax.lax.dot_general(
        h, w2_ref[...], (((1,), (0,)), ((), ())),
        preferred_element_type=jnp.float32,
        precision=jax.lax.Precision.HIGHEST)

  return pl.pallas_call(
      body,
      grid=(grid,),
      in_specs=[
          pl.BlockSpec((NC_, blk, 16), lambda i: (0, i, 0)),
          pl.BlockSpec((NC_, blk, 16), lambda i: (0, i, 0)),
          pl.BlockSpec((16, blk), lambda i: (0, i)),
          pl.BlockSpec((1, 16), lambda i: (0, 0)),
          pl.BlockSpec((16, 128), lambda i: (0, 0)),
      ],
      out_specs=[
          pl.BlockSpec((blk, 16), lambda i: (i, 0)),
          pl.BlockSpec((blk, 128), lambda i: (i, 0)),
          pl.BlockSpec((blk, 16), lambda i: (i, 0)),
      ],
      out_shape=[
          jax.ShapeDtypeStruct((n, 16), jnp.float32),
          jax.ShapeDtypeStruct((n, 128), jnp.float32),
          jax.ShapeDtypeStruct((n, 16), jnp.float32),
      ],
  )(degp, s1p, r1t, bias1, w2cat)


def _tc_layer2(s2p, inv16, h, root2, bias2, n, blk):
  grid = -(-n // blk)

  def body(s2p_ref, inv_ref, h_ref, root2_ref, bias2_ref, out_ref):
    z = (s2p_ref[0] + s2p_ref[1]) * inv_ref[...]
    z = z + jax.lax.dot_general(
        h_ref[...], root2_ref[...], (((1,), (0,)), ((), ())),
        preferred_element_type=jnp.float32,
        precision=jax.lax.Precision.HIGHEST)
    z = z + bias2_ref[...]
    m = jnp.max(z, axis=1, keepdims=True)
    e = jnp.exp(z - m)
    out_ref[...] = z - m - jnp.log(jnp.sum(e, axis=1, keepdims=True))

  return pl.pallas_call(
      body,
      grid=(grid,),
      in_specs=[
          pl.BlockSpec((NC_, blk, 16), lambda i: (0, i, 0)),
          pl.BlockSpec((blk, 16), lambda i: (i, 0)),
          pl.BlockSpec((blk, 16), lambda i: (i, 0)),
          pl.BlockSpec((16, 16), lambda i: (0, 0)),
          pl.BlockSpec((1, 16), lambda i: (0, 0)),
      ],
      out_specs=pl.BlockSpec((blk, 16), lambda i: (i, 0)),
      out_shape=jax.ShapeDtypeStruct((n, 16), jnp.float32),
  )(s2p, inv16, h, root2, bias2)


def kernel(edge_index, edge_type, edge_norm, weight1, root1, bias1,
           weight2, root2, bias2):
  del edge_norm
  r, n, c1 = weight1.shape
  nc2 = weight2.shape[2]
  e = edge_type.shape[0]

  e_pad = NW_ * CPT_ * CH_
  npad = -(-(n + 1) // CH_) * CH_
  ndum = npad - n

  del ndum
  blk = 2048
  ei2 = edge_index.astype(jnp.int32).reshape(2, e // CH_, CH_)
  et2 = edge_type.astype(jnp.int32).reshape(e // CH_, CH_)
  idxs, dstp = _tc_indices(ei2, et2, r, n, npad, e_pad // CH_, 256)

  degp = _sc_degree(dstp, npad)

  w128 = jnp.transpose(weight1, (0, 2, 1)).reshape(r * c1, n)
  table1 = _tc_table1(w128, r * c1, n, blk).reshape(n * r, c1)
  s1p = _sc_gather_add(table1, idxs, dstp, npad)

  w2cat = jnp.transpose(weight2, (1, 0, 2)).reshape(c1, r * nc2)
  r1t = jnp.transpose(root1, (1, 0))
  h, t2, inv16 = _tc_layer1(degp, s1p, r1t, bias1.reshape(1, 16), w2cat,
                            n, npad, blk)

  table2 = t2.reshape(n * r, nc2)
  s2p = _sc_gather_add(table2, idxs, dstp, npad)

  return _tc_layer2(s2p, inv16, h, root2, bias2.reshape(1, 16), n, blk)


# speedup vs baseline: 81.1843x; 1.2240x over previous
import functools

import jax
import jax.numpy as jnp
from jax import lax
from jax.experimental import pallas as pl
from jax.experimental.pallas import tpu as pltpu
from jax.experimental.pallas import tpu_sc as plsc

NC_ = 2
NS_ = 16
NW_ = NC_ * NS_
CH_ = 128

G_ = 4
GPS_ = 14
NSB_ = 7
CPT_ = G_ * GPS_ * NSB_

_SC_PARAMS = pltpu.CompilerParams(use_tc_tiling_on_sc=False)


def _zero_rows(buf, nrows):
  @pl.loop(0, nrows)
  def _(i):
    buf[i, :] = jnp.zeros((16,), jnp.float32)


def _zero_acc_slice(acc, zsrc, base, rpt):
  zrows = zsrc.shape[0]
  nfull, tail = divmod(rpt, zrows)

  @pl.loop(0, nfull)
  def _(i):
    pltpu.sync_copy(zsrc, acc.at[pl.ds(base + i * zrows, zrows)])

  if tail:
    pltpu.sync_copy(zsrc.at[pl.ds(0, tail)],
                    acc.at[pl.ds(base + nfull * zrows, tail)])


def _sc_gather_add(table, idx2d, dst2d, npad):
  rpt = npad // NS_
  spc = G_ * GPS_
  pcc = CPT_ * NS_
  grows = G_ * CH_
  mesh = plsc.VectorSubcoreMesh(core_axis_name="c", subcore_axis_name="s")

  @functools.partial(
      pl.kernel,
      out_type=jax.ShapeDtypeStruct((NC_, npad, 16), jnp.float32),
      mesh=mesh,
      scratch_types=[
          pltpu.VMEM_SHARED((npad, 16), jnp.float32),
          pltpu.VMEM((spc, CH_), jnp.int32),
          pltpu.VMEM((spc, CH_), jnp.int32),
          pltpu.VMEM((grows, 16), jnp.float32),
          pltpu.VMEM((grows, 16), jnp.float32),
          pltpu.SemaphoreType.DMA,
          pltpu.SemaphoreType.DMA,
          pltpu.SemaphoreType.DMA,
      ],
      compiler_params=_SC_PARAMS,
  )
  def k(idx_hbm, dst_hbm, table_hbm, out_hbm, acc, idxa, dsta, rows0, rows1,
        lsem, gsem, ssem):
    c = lax.axis_index("c")
    s = lax.axis_index("s")
    rows = (rows0, rows1)

    _zero_rows(rows0, grows)
    base = s * rpt
    _zero_acc_slice(acc, rows0, base, rpt)
    plsc.subcore_barrier()

    chunk0 = c * pcc + s * CPT_

    def issue_gathers(t, b):
      for j in range(G_):
        pltpu.async_copy(table_hbm.at[idxa.at[t * G_ + j]],
                         rows[b].at[pl.ds(j * CH_, CH_)], gsem)

    def issue_scatters(t, b):
      for j in range(G_):
        pltpu.async_copy(rows[b].at[pl.ds(j * CH_, CH_)],
                         acc.at[dsta.at[t * G_ + j]], ssem, add=True)

    def drain_g(b):
      pltpu.make_async_copy(table_hbm.at[pl.ds(0, grows)], rows[b], gsem).wait()

    def drain_s(b):
      pltpu.make_async_copy(rows[b], acc.at[pl.ds(0, grows)], ssem).wait()

    @pl.loop(0, NSB_)
    def _(sb):
      cb = chunk0 + sb * spc
      cpi = pltpu.async_copy(idx_hbm.at[pl.ds(cb, spc)], idxa, lsem)
      cpd = pltpu.async_copy(dst_hbm.at[pl.ds(cb, spc)], dsta, lsem)
      cpi.wait()
      cpd.wait()

      @pl.loop(0, GPS_, step=2)
      def _(base_t):
        for b2 in (0, 1):
          t = base_t + b2
          if b2 == 0:
            @pl.when(base_t > 0)
            def _():
              drain_g(1)
              issue_scatters(t - 1, 1)
              drain_s(0)
          else:
            drain_g(0)
            issue_scatters(t - 1, 0)

            @pl.when(base_t > 0)
            def _():
              drain_s(1)
          issue_gathers(t, b2)

      drain_g(1)
      issue_scatters(GPS_ - 1, 1)
      drain_s(0)
      drain_s(1)

    plsc.subcore_barrier()
    pltpu.sync_copy(acc.at[pl.ds(base, rpt)], out_hbm.at[c, pl.ds(base, rpt)])

  return k(idx2d, dst2d, table)


def _sc_degree(dst2d, npad):
  rpt = npad // NS_
  pcc = CPT_ * NS_
  grows = G_ * CH_
  mesh = plsc.VectorSubcoreMesh(core_axis_name="c", subcore_axis_name="s")

  @functools.partial(
      pl.kernel,
      out_type=jax.ShapeDtypeStruct((NC_, npad, 16), jnp.float32),
      mesh=mesh,
      scratch_types=[
          pltpu.VMEM_SHARED((npad, 16), jnp.float32),
          pltpu.VMEM((G_ * GPS_, CH_), jnp.int32),
          pltpu.VMEM((grows, 16), jnp.float32),
          pltpu.SemaphoreType.DMA,
          pltpu.SemaphoreType.DMA,
      ],
      compiler_params=_SC_PARAMS,
  )
  def k(dst_hbm, out_hbm, acc, dsta, onesb, lsem, ssem):
    c = lax.axis_index("c")
    s = lax.axis_index("s")
    spc = G_ * GPS_

    _zero_rows(onesb, grows)
    base = s * rpt
    _zero_acc_slice(acc, onesb, base, rpt)

    @pl.loop(0, grows)
    def _(i):
      onesb[i, :] = jnp.full((16,), 1.0, jnp.float32)

    plsc.subcore_barrier()
    chunk0 = c * pcc + s * CPT_

    def drain_s():
      pltpu.make_async_copy(onesb, acc.at[pl.ds(0, grows)], ssem).wait()

    @pl.loop(0, NSB_)
    def _(sb):
      pltpu.async_copy(dst_hbm.at[pl.ds(chunk0 + sb * spc, spc)], dsta,
                       lsem).wait()

      @pl.loop(0, GPS_)
      def _(t):
        @pl.when(t >= 2)
        def _():
          drain_s()
        for j in range(G_):
          pltpu.async_copy(onesb.at[pl.ds(j * CH_, CH_)],
                           acc.at[dsta.at[t * G_ + j]], ssem, add=True)

      drain_s()
      drain_s()

    plsc.subcore_barrier()
    pltpu.sync_copy(acc.at[pl.ds(base, rpt)], out_hbm.at[c, pl.ds(base, rpt)])

  return k(dst2d)


def _tc_table1(w128, rc, n, blk):
  grid = -(-n // blk)

  def body(w_ref, t_ref):
    t_ref[...] = jnp.transpose(w_ref[...], (1, 0))

  return pl.pallas_call(
      body,
      grid=(grid,),
      in_specs=[pl.BlockSpec((rc, blk), lambda i: (0, i))],
      out_specs=pl.BlockSpec((blk, rc), lambda i: (i, 0)),
      out_shape=jax.ShapeDtypeStruct((n, rc), jnp.float32),
  )(w128)


def _tc_indices(ei2, et2, r, n, npad, nchunk_pad, blk):
  grid = nchunk_pad // blk
  nchunk = ei2.shape[1]

  def body(ei_ref, et_ref, idx_ref, dst_ref):
    i = pl.program_id(0)
    row = jax.lax.broadcasted_iota(jnp.int32, (blk, 128), 0) + i * blk
    lane = jax.lax.broadcasted_iota(jnp.int32, (blk, 128), 1)
    gid = row * 128 + lane
    real = row < nchunk
    idx_ref[...] = jnp.where(real, ei_ref[0] * r + et_ref[...], gid % n)
    dst_ref[...] = jnp.where(real, ei_ref[1], n + gid % (npad - n))

  return pl.pallas_call(
      body,
      grid=(grid,),
      in_specs=[
          pl.BlockSpec((2, blk, 128), lambda i: (0, i, 0)),
          pl.BlockSpec((blk, 128), lambda i: (i, 0)),
      ],
      out_specs=[
          pl.BlockSpec((blk, 128), lambda i: (i, 0)),
          pl.BlockSpec((blk, 128), lambda i: (i, 0)),
      ],
      out_shape=[
          jax.ShapeDtypeStruct((nchunk_pad, 128), jnp.int32),
          jax.ShapeDtypeStruct((nchunk_pad, 128), jnp.int32),
      ],
  )(ei2, et2)


def _tc_layer1(degb, s1b, r1pk, b1pk, w2big, nwp, nw, wb):
  grid = nwp // wb

  def body(degb_ref, s1b_ref, r1_ref, b1_ref, w2_ref,
           h_ref, t2_ref, inv_ref):
    deg = degb_ref[0] + degb_ref[1]
    inv = 1.0 / jnp.maximum(deg, 1.0)
    s1 = (s1b_ref[0] + s1b_ref[1]) * inv
    h = jnp.maximum(s1 + r1_ref[...] + b1_ref[...], 0.0)
    h_ref[...] = h
    inv_ref[...] = inv
    t2_ref[...] = jax.lax.dot_general(
        h, w2_ref[...], (((1,), (0,)), ((), ())),
        preferred_element_type=jnp.float32,
        precision=jax.lax.Precision.HIGHEST)

  return pl.pallas_call(
      body,
      grid=(grid,),
      in_specs=[
          pl.BlockSpec((NC_, wb, 128), lambda i: (0, i, 0)),
          pl.BlockSpec((NC_, wb, 128), lambda i: (0, i, 0)),
          pl.BlockSpec((wb, 128), lambda i: (i, 0)),
          pl.BlockSpec((1, 128), lambda i: (0, 0)),
          pl.BlockSpec((128, 1024), lambda i: (0, 0)),
      ],
      out_specs=[
          pl.BlockSpec((wb, 128), lambda i: (i, 0)),
          pl.BlockSpec((wb, 1024), lambda i: (i, 0)),
          pl.BlockSpec((wb, 128), lambda i: (i, 0)),
      ],
      out_shape=[
          jax.ShapeDtypeStruct((nwp, 128), jnp.float32),
          jax.ShapeDtypeStruct((nwp, 1024), jnp.float32),
          jax.ShapeDtypeStruct((nwp, 128), jnp.float32),
      ],
  )(degb, s1b, r1pk, b1pk, w2big)


def _tc_layer2(s2b, invb, hpk, r2big, b2pk, nw, wb):
  grid = -(-nw // wb)

  def body(s2b_ref, inv_ref, h_ref, r2_ref, b2_ref, out_ref):
    z = (s2b_ref[0] + s2b_ref[1]) * inv_ref[...]
    z = z + jax.lax.dot_general(
        h_ref[...], r2_ref[...], (((1,), (0,)), ((), ())),
        preferred_element_type=jnp.float32,
        precision=jax.lax.Precision.HIGHEST)
    z = z + b2_ref[...]
    off = jax.lax.broadcasted_iota(jnp.int32, (wb, 128), 1) % 16

    def segshift(x, sh):
      a = pltpu.roll(x, 128 - sh, 1)
      b = pltpu.roll(x, 16 - sh, 1)
      return jnp.where(off < 16 - sh, a, b)

    m = z
    for sh in (1, 2, 4, 8):
      m = jnp.maximum(m, segshift(m, sh))
    e = jnp.exp(z - m)
    ssum = e
    for sh in (1, 2, 4, 8):
      ssum = ssum + segshift(ssum, sh)
    out_ref[...] = z - m - jnp.log(ssum)

  return pl.pallas_call(
      body,
      grid=(grid,),
      in_specs=[
          pl.BlockSpec((NC_, wb, 128), lambda i: (0, i, 0)),
          pl.BlockSpec((wb, 128), lambda i: (i, 0)),
          pl.BlockSpec((wb, 128), lambda i: (i, 0)),
          pl.BlockSpec((128, 128), lambda i: (0, 0)),
          pl.BlockSpec((1, 128), lambda i: (0, 0)),
      ],
      out_specs=pl.BlockSpec((wb, 128), lambda i: (i, 0)),
      out_shape=jax.ShapeDtypeStruct((nw, 128), jnp.float32),
  )(s2b, invb, hpk, r2big, b2pk)


def kernel(edge_index, edge_type, edge_norm, weight1, root1, bias1,
           weight2, root2, bias2):
  del edge_norm
  r, n, c1 = weight1.shape
  nc2 = weight2.shape[2]
  e = edge_type.shape[0]

  e_pad = NW_ * CPT_ * CH_
  npad = -(-(n + 1) // CH_) * CH_
  ndum = npad - n

  del ndum
  blk = 2048
  ei2 = edge_index.astype(jnp.int32).reshape(2, e // CH_, CH_)
  et2 = edge_type.astype(jnp.int32).reshape(e // CH_, CH_)
  idxs, dstp = _tc_indices(ei2, et2, r, n, npad, e_pad // CH_, 256)

  degp = _sc_degree(dstp, npad)

  w128 = jnp.transpose(weight1, (0, 2, 1)).reshape(r * c1, n)
  table1 = _tc_table1(w128, r * c1, n, blk).reshape(n * r, c1)
  s1p = _sc_gather_add(table1, idxs, dstp, npad)

  nw = n // 8
  nwp = nw + 44
  wb = 256
  w2cat = jnp.transpose(weight2, (1, 0, 2)).reshape(c1, r * nc2)
  w2big = jnp.kron(jnp.eye(8, dtype=jnp.float32), w2cat)
  r2big = jnp.kron(jnp.eye(8, dtype=jnp.float32), root2)
  b1pk = jnp.tile(bias1, 8).reshape(1, 128)
  b2pk = jnp.tile(bias2, 8).reshape(1, 128)
  r1pk = root1.reshape(nw, 128)
  degb = degp.reshape(NC_, npad // 8, 128)
  s1b = s1p.reshape(NC_, npad // 8, 128)
  hpk, t2pk, invb = _tc_layer1(degb, s1b, r1pk, b1pk, w2big, nwp, nw, wb)

  table2 = t2pk.reshape(nwp * 64, nc2)
  s2p = _sc_gather_add(table2, idxs, dstp, npad)

  outpk = _tc_layer2(s2p.reshape(NC_, npad // 8, 128), invb, hpk, r2big,
                     b2pk, nw, wb)
  return outpk.reshape(n, 16)
